# Initial kernel scaffold; baseline (speedup 1.0000x reference)
import jax
import jax.numpy as jnp
from jax.experimental import pallas as pl


def kernel(input_ids, table):
    raise NotImplementedError("write your pallas kernel here")


# speedup vs baseline: 1.9906x; 1.9906x over previous
import functools

import jax
import jax.numpy as jnp
import numpy as np
from jax import lax
from jax.experimental import pallas as pl
from jax.experimental.pallas import tpu as pltpu
from jax.experimental.pallas import tpu_sc as plsc

HEADS = 4
TABLE = 1000000
DIM = 64
PRIME = (31, 37, 41, 43)
BATCH = 1024
SEQ = 200
N = BATCH * SEQ
CHUNK = 128
ROWS = N // CHUNK
NW = 32
RPW = ROWS // NW


def _mod_const(n, m):
    q = jnp.floor(n.astype(jnp.float32) * (1.0 / m)).astype(jnp.int32)
    r = n - q * m
    r = jnp.where(r < 0, r + m, r)
    r = jnp.where(r >= m, r - m, r)
    return r


def _hash_body(x0_ref, x1_ref, x2_ref, out_ref):
    x0 = x0_ref[...]
    x1 = x1_ref[...]
    x2 = x2_ref[...]
    w = lax.broadcasted_iota(jnp.int32, (NW, RPW, CHUNK), 0)
    r = lax.broadcasted_iota(jnp.int32, (NW, RPW, CHUNK), 1)
    c = lax.broadcasted_iota(jnp.int32, (NW, RPW, CHUNK), 2)
    s = _mod_const((w * RPW + r) * CHUNK + c, SEQ)
    m3 = s >= 2
    m2 = s >= 1
    for h in range(HEADS):
        p = PRIME[h]
        h3 = _mod_const(x2 ^ (x1 * p) ^ (x0 * (p * p)), TABLE)
        h3 = jnp.where(m3, h3, 0)
        h2 = _mod_const(x1 ^ (x0 * p), TABLE)
        h2 = jnp.where(m2, h2, 0)
        out_ref[:, :, h * CHUNK:(h + 1) * CHUNK] = _mod_const(h3 ^ h2, TABLE)


_hash_call = pl.pallas_call(
    _hash_body,
    out_shape=jax.ShapeDtypeStruct((NW, RPW, HEADS * CHUNK), jnp.int32),
)


@functools.cache
def _make_sc_gather():
    mesh = plsc.VectorSubcoreMesh(core_axis_name="c", subcore_axis_name="s")

    @functools.partial(
        pl.kernel,
        mesh=mesh,
        out_type=jax.ShapeDtypeStruct((ROWS, CHUNK, DIM), jnp.float32),
        scratch_types=[
            pltpu.VMEM((RPW * HEADS, CHUNK), jnp.int32),
            pltpu.VMEM((2, HEADS, CHUNK, DIM), jnp.float32),
            pltpu.VMEM((2, CHUNK, DIM), jnp.float32),
            pltpu.SemaphoreType.DMA,
            pltpu.SemaphoreType.DMA,
            pltpu.SemaphoreType.DMA,
        ],
        compiler_params=pltpu.CompilerParams(use_tc_tiling_on_sc=False),
    )
    def _sc_gather(idx_hbm, table_hbm, out_hbm, idx_v, rows, out_v,
                   semg0, semg1, semo):
        wid = lax.axis_index("s") * np.int32(2) + lax.axis_index("c")
        row0 = wid * np.int32(RPW)
        pltpu.async_copy(idx_hbm.at[wid], idx_v, semo).wait()
        gsems = (semg0, semg1)

        def start_gather(j, b):
            for h in range(HEADS):
                pltpu.async_copy(
                    table_hbm.at[idx_v.at[j * np.int32(HEADS) + np.int32(h)]],
                    rows.at[np.int32(b), np.int32(h)], gsems[b])

        def wait_gather(j, b):
            for h in range(HEADS):
                pltpu.make_async_copy(
                    table_hbm.at[idx_v.at[j * np.int32(HEADS) + np.int32(h)]],
                    rows.at[np.int32(b), np.int32(h)], gsems[b]).wait()

        def compute_and_store(j, b):
            bb = np.int32(b)

            def pos_body(_, p):
                for k in range(DIM // 16):
                    sl = pl.ds(k * 16, 16)
                    v = ((rows[bb, np.int32(0), p, sl]
                          + rows[bb, np.int32(1), p, sl])
                         + (rows[bb, np.int32(2), p, sl]
                            + rows[bb, np.int32(3), p, sl]))
                    out_v[bb, p, sl] = v
                return p + np.int32(1)

            lax.fori_loop(0, CHUNK, pos_body, np.int32(0), unroll=4)
            pltpu.async_copy(out_v.at[bb], out_hbm.at[row0 + j], semo).wait()

        start_gather(np.int32(0), 0)

        def body(_, j):
            start_gather(j + np.int32(1), 1)
            wait_gather(j, 0)
            compute_and_store(j, 0)
            start_gather(j + np.int32(2), 0)
            wait_gather(j + np.int32(1), 1)
            compute_and_store(j + np.int32(1), 1)
            return j + np.int32(2)

        jlast = lax.fori_loop(0, RPW // 2 - 1, body, np.int32(0))
        start_gather(jlast + np.int32(1), 1)
        wait_gather(jlast, 0)
        compute_and_store(jlast, 0)
        wait_gather(jlast + np.int32(1), 1)
        compute_and_store(jlast + np.int32(1), 1)

    return _sc_gather


def kernel(input_ids, table):
    ids = input_ids.astype(jnp.int32).reshape(-1)
    x1 = jnp.concatenate([jnp.zeros((1,), jnp.int32), ids[:-1]])
    x2 = jnp.concatenate([jnp.zeros((2,), jnp.int32), ids[:-2]])
    shp = (NW, RPW, CHUNK)
    idx = _hash_call(ids.reshape(shp), x1.reshape(shp), x2.reshape(shp))
    idx = idx.reshape(NW, RPW * HEADS, CHUNK)
    out = _make_sc_gather()(idx, table)
    return out.reshape(BATCH, SEQ, DIM)
